# Initial kernel scaffold; baseline (speedup 1.0000x reference)
import jax
import jax.numpy as jnp
from jax.experimental import pallas as pl


def kernel(basefeatureMap, imageInfo, groundTruthBoxes, numBoxes, W_conv, b_conv, W_cls, b_cls, W_bbox, b_bbox):
    raise NotImplementedError("write your pallas kernel here")


# speedup vs baseline: 1.5035x; 1.5035x over previous
import functools

import numpy as np
import jax
import jax.numpy as jnp
from jax import lax
from jax.experimental import pallas as pl

B, DIN, H, W = 2, 256, 50, 76
FEAT_STRIDE = 16
PRE_NMS = 2000
POST_NMS = 128
NMS_TH = 0.7
MIN_SIZE = 16.0

WP = W + 2
NQ = H * WP
NQP = 3904
NA = 9
NPAD = 2048


def _np_mkanchors(ws, hs, x_ctr, y_ctr):
    ws = ws[:, None]
    hs = hs[:, None]
    return np.hstack((x_ctr - 0.5 * (ws - 1), y_ctr - 0.5 * (hs - 1),
                      x_ctr + 0.5 * (ws - 1), y_ctr + 0.5 * (hs - 1)))


def _np_generate_anchors(base_size=16, ratios=(0.5, 1.0, 2.0), scales=(8.0, 16.0, 32.0)):
    ratios = np.array(ratios, dtype=np.float64)
    scales = np.array(scales, dtype=np.float64)
    w = float(base_size); h = float(base_size)
    x_ctr = 0.5 * (w - 1); y_ctr = 0.5 * (h - 1)
    size = w * h
    ws = np.round(np.sqrt(size / ratios))
    hs = np.round(ws * ratios)
    ratio_anchors = _np_mkanchors(ws, hs, x_ctr, y_ctr)
    out = []
    for ra in ratio_anchors:
        w2 = ra[2] - ra[0] + 1; h2 = ra[3] - ra[1] + 1
        xc = ra[0] + 0.5 * (w2 - 1); yc = ra[1] + 0.5 * (h2 - 1)
        out.append(_np_mkanchors(w2 * scales, h2 * scales, xc, yc))
    return np.vstack(out).astype(np.float32)

_ANCHORS = _np_generate_anchors()
_AW = (_ANCHORS[:, 2] - _ANCHORS[:, 0] + 1.0)
_AH = (_ANCHORS[:, 3] - _ANCHORS[:, 1] + 1.0)
_ACX = (_ANCHORS[:, 0] + 0.5 * _AW)
_ACY = (_ANCHORS[:, 1] + 0.5 * _AH)

_QS = np.arange(NQP)
_YS = _QS // WP
_XS = _QS % WP
_SX = (_XS * FEAT_STRIDE).astype(np.float32).reshape(1, NQP)
_SY = (_YS * FEAT_STRIDE).astype(np.float32).reshape(1, NQP)
_GM = ((_XS >= W) | (_YS >= H)).astype(np.float32).reshape(1, NQP)


RT = 488


def _stage1_body(x_ref, w1_ref, b1_ref, w2_ref, b2_ref, out_ref):
    cb = pl.program_id(1)
    for r in range(NQP // RT):
        base = r * RT
        acc = jnp.zeros((RT, 128), jnp.float32)
        for t in range(9):
            dy, dx = t // 3, t % 3
            a = x_ref[0, pl.ds(base + dy * WP + dx, RT), :]
            acc = acc + lax.dot_general(a, w1_ref[t], (((1,), (0,)), ((), ())),
                                        preferred_element_type=jnp.float32)
        h = jnp.maximum(acc + b1_ref[0], 0.0)
        contrib = lax.dot_general(h, w2_ref[:], (((1,), (0,)), ((), ())),
                                  preferred_element_type=jnp.float32)

        @pl.when(cb == 0)
        def _():
            out_ref[0, pl.ds(base, RT), :] = jnp.broadcast_to(b2_ref[:], (RT, 64))

        out_ref[0, pl.ds(base, RT), :] = out_ref[0, pl.ds(base, RT), :] + contrib


def _decode_body(o_ref, sx_ref, sy_ref, gm_ref, cl_ref, sc_ref, bx_ref):
    o = o_ref[0]
    sx = sx_ref[:]
    sy = sy_ref[:]
    gm = gm_ref[:]
    cl = cl_ref[0]
    maxx = cl[0:1, 0:1]
    maxy = cl[0:1, 1:2]
    minsz = cl[0:1, 2:3]
    for a in range(NA):
        s0 = o[a:a + 1, :]
        s1 = o[9 + a:10 + a, :]
        m = jnp.maximum(s0, s1)
        e0 = jnp.exp(s0 - m)
        e1 = jnp.exp(s1 - m)
        p = e1 / (e0 + e1)
        dxv = o[18 + a:19 + a, :]
        dyv = o[27 + a:28 + a, :]
        dwv = o[36 + a:37 + a, :]
        dhv = o[45 + a:46 + a, :]
        wa = float(_AW[a]); ha = float(_AH[a])
        cx = sx + float(_ACX[a])
        cy = sy + float(_ACY[a])
        pcx = dxv * wa + cx
        pcy = dyv * ha + cy
        pw = jnp.exp(dwv) * wa
        ph = jnp.exp(dhv) * ha
        x1 = pcx - 0.5 * pw
        y1 = pcy - 0.5 * ph
        x2 = pcx + 0.5 * pw
        y2 = pcy + 0.5 * ph
        x1 = jnp.minimum(jnp.maximum(x1, 0.0), maxx)
        y1 = jnp.minimum(jnp.maximum(y1, 0.0), maxy)
        x2 = jnp.minimum(jnp.maximum(x2, 0.0), maxx)
        y2 = jnp.minimum(jnp.maximum(y2, 0.0), maxy)
        ws = x2 - x1 + 1.0
        hs = y2 - y1 + 1.0
        ok = (ws >= minsz) & (hs >= minsz)
        sc = jnp.where(ok, p, -1.0)
        sc = jnp.where(gm > 0.0, -jnp.inf, sc)
        sc_ref[0, a:a + 1, :] = sc
        bx_ref[0, 4 * a:4 * a + 1, :] = x1
        bx_ref[0, 4 * a + 1:4 * a + 2, :] = y1
        bx_ref[0, 4 * a + 2:4 * a + 3, :] = x2
        bx_ref[0, 4 * a + 3:4 * a + 4, :] = y2


def _nms_body(s_ref, pl_ref, brow_ref, out_ref):
    bf = pl.program_id(0).astype(jnp.float32)
    x1p = pl_ref[0, 0]
    y1p = pl_ref[0, 1]
    x2p = pl_ref[0, 2]
    y2p = pl_ref[0, 3]
    areas = (x2p - x1p + 1.0) * (y2p - y1p + 1.0)
    flat = (lax.broadcasted_iota(jnp.int32, (16, 128), 0) * 128
            + lax.broadcasted_iota(jnp.int32, (16, 128), 1))

    def step(i, s):
        m = jnp.max(s)
        j = jnp.min(jnp.where(s == m, flat, NPAD))
        j = jnp.where(m == -jnp.inf, 0, j)
        sel = flat == j
        bx1 = jnp.sum(jnp.where(sel, x1p, 0.0))
        by1 = jnp.sum(jnp.where(sel, y1p, 0.0))
        bx2 = jnp.sum(jnp.where(sel, x2p, 0.0))
        by2 = jnp.sum(jnp.where(sel, y2p, 0.0))
        aj = jnp.sum(jnp.where(sel, areas, 0.0))
        row = jnp.reshape(jnp.stack([bf, bx1, by1, bx2, by2, 0.0, 0.0, 0.0]), (1, 1, 8))
        out_ref[0:1, pl.ds(i, 1), :] = row
        xx1 = jnp.maximum(bx1, x1p)
        yy1 = jnp.maximum(by1, y1p)
        xx2 = jnp.minimum(bx2, x2p)
        yy2 = jnp.minimum(by2, y2p)
        iw = jnp.maximum(0.0, xx2 - xx1 + 1.0)
        ih = jnp.maximum(0.0, yy2 - yy1 + 1.0)
        inter = iw * ih
        iou = inter / (aj + areas - inter)
        return jnp.where(iou > NMS_TH, -jnp.inf, s)

    lax.fori_loop(0, POST_NMS, step, s_ref[0], unroll=False)


def kernel(basefeatureMap, imageInfo, groundTruthBoxes, numBoxes,
           W_conv, b_conv, W_cls, b_cls, W_bbox, b_bbox):
    f32 = jnp.float32
    x = jnp.transpose(basefeatureMap, (0, 2, 3, 1))
    x = jnp.pad(x, ((0, 0), (1, 1), (1, 1), (0, 0)))
    x = x.reshape(B, 52 * WP, DIN)
    x = jnp.pad(x, ((0, 0), (0, 4064 - 52 * WP), (0, 0)))
    W1 = jnp.transpose(W_conv, (2, 3, 1, 0)).reshape(9, DIN, 512)
    b1 = b_conv.reshape(4, 1, 128)
    Wc = W_cls[:, :, 0, 0]
    Wb = W_bbox[:, :, 0, 0]
    perm = np.concatenate([np.arange(9) * 4 + j for j in range(4)])
    W2 = jnp.concatenate([Wc.T, Wb.T[:, perm]], axis=1)
    W2 = jnp.pad(W2, ((0, 0), (0, 10)))
    b2 = jnp.pad(jnp.concatenate([b_cls, b_bbox[perm]]), (0, 10)).reshape(1, 64)

    out2 = pl.pallas_call(
        _stage1_body,
        grid=(B, 4),
        in_specs=[
            pl.BlockSpec((1, 4064, DIN), lambda b, c: (b, 0, 0)),
            pl.BlockSpec((9, DIN, 128), lambda b, c: (0, 0, c)),
            pl.BlockSpec((1, 1, 128), lambda b, c: (c, 0, 0)),
            pl.BlockSpec((128, 64), lambda b, c: (c, 0)),
            pl.BlockSpec((1, 64), lambda b, c: (0, 0)),
        ],
        out_specs=pl.BlockSpec((1, NQP, 64), lambda b, c: (b, 0, 0)),
        out_shape=jax.ShapeDtypeStruct((B, NQP, 64), f32),
    )(x, W1, b1, W2, b2)
    out2T = jnp.transpose(out2, (0, 2, 1))

    sxc = jnp.asarray(_SX)
    syc = jnp.asarray(_SY)
    gmc = jnp.asarray(_GM)
    cl = jnp.stack([imageInfo[:, 1] - 1.0, imageInfo[:, 0] - 1.0,
                    MIN_SIZE * imageInfo[:, 2], jnp.zeros((B,), f32)], axis=1)
    cl = cl.reshape(B, 1, 4)

    scores_t, boxes_t = pl.pallas_call(
        _decode_body,
        grid=(B,),
        in_specs=[
            pl.BlockSpec((1, 64, NQP), lambda b: (b, 0, 0)),
            pl.BlockSpec((1, NQP), lambda b: (0, 0)),
            pl.BlockSpec((1, NQP), lambda b: (0, 0)),
            pl.BlockSpec((1, NQP), lambda b: (0, 0)),
            pl.BlockSpec((1, 1, 4), lambda b: (b, 0, 0)),
        ],
        out_specs=[
            pl.BlockSpec((1, NA, NQP), lambda b: (b, 0, 0)),
            pl.BlockSpec((1, 4 * NA, NQP), lambda b: (b, 0, 0)),
        ],
        out_shape=[
            jax.ShapeDtypeStruct((B, NA, NQP), f32),
            jax.ShapeDtypeStruct((B, 4 * NA, NQP), f32),
        ],
    )(out2T, sxc, syc, gmc, cl)

    nflat = NQP * NA
    scores_f = jnp.transpose(scores_t, (0, 2, 1)).reshape(B, nflat)
    boxes_f = jnp.transpose(boxes_t, (0, 2, 1)).reshape(B, nflat, 4)

    top_s, top_i = lax.top_k(scores_f, PRE_NMS)
    top_b = jnp.take_along_axis(boxes_f, top_i[:, :, None], axis=1)

    sc2 = jnp.pad(top_s, ((0, 0), (0, NPAD - PRE_NMS)),
                  constant_values=-jnp.inf).reshape(B, 16, 128)
    bb2 = jnp.pad(top_b, ((0, 0), (0, NPAD - PRE_NMS), (0, 0)))
    planes = jnp.transpose(bb2, (0, 2, 1)).reshape(B, 4, 16, 128)

    rois8 = pl.pallas_call(
        _nms_body,
        grid=(B,),
        in_specs=[
            pl.BlockSpec((1, 16, 128), lambda b: (b, 0, 0)),
            pl.BlockSpec((1, 4, 16, 128), lambda b: (b, 0, 0, 0)),
            pl.BlockSpec((1, NPAD, 4), lambda b: (b, 0, 0)),
        ],
        out_specs=pl.BlockSpec((1, POST_NMS, 8), lambda b: (b, 0, 0)),
        out_shape=jax.ShapeDtypeStruct((B, POST_NMS, 8), f32),
    )(sc2, planes, bb2)

    rois = rois8[:, :, :5]
    return (rois, jnp.zeros((), f32), jnp.zeros((), f32))
